# Initial kernel scaffold; baseline (speedup 1.0000x reference)
import jax
import jax.numpy as jnp
from jax.experimental import pallas as pl


def kernel(node_features, edge_features, global_features, edge_list, W_skip, b_skip, W_src, b_src, W_dst, b_dst, W_edge, b_edge, attn_a, ln1_scale, ln1_bias, Wq, bq, Wk, bk, Wv, bv, Wo, bo, global_param, lnf_scale, lnf_bias):
    raise NotImplementedError("write your pallas kernel here")


# speedup vs baseline: 1.0234x; 1.0234x over previous
import functools

import jax
import jax.numpy as jnp
from jax.experimental import pallas as pl
from jax.experimental.pallas import tpu as pltpu

N = 10000
E = 320000
EMBED = 256
HEADS = 8
DH = 32
LAYERS = 4

NB = 1000
EB = 8000
_INV_SQRT_DH = 1.0 / (DH ** 0.5)


def _ka_body(nf_ref, gf_ref, wsk_ref, bsk_ref, wsr_ref, bsr_ref, wd_ref,
             bd_ref, skip_ref, xs_ref, xd_ref):
    nf = nf_ref[...]
    skip = jnp.dot(nf, wsk_ref[...], preferred_element_type=jnp.float32) + bsk_ref[...]
    nfr = jnp.maximum(skip + gf_ref[...], 0.0)
    skip_ref[...] = skip
    xs_ref[...] = jnp.dot(nfr, wsr_ref[...], preferred_element_type=jnp.float32) + bsr_ref[...]
    xd_ref[...] = jnp.dot(nfr, wd_ref[...], preferred_element_type=jnp.float32) + bd_ref[...]


def _proj_nodes(nf, gf, wsk, bsk, wsr, bsr, wd, bd):
    full = lambda i: (0, 0)
    blk = lambda i: (i, 0)
    return pl.pallas_call(
        _ka_body,
        grid=(N // NB,),
        in_specs=[
            pl.BlockSpec((NB, EMBED), blk),
            pl.BlockSpec((1, EMBED), full),
            pl.BlockSpec((EMBED, EMBED), full),
            pl.BlockSpec((1, EMBED), full),
            pl.BlockSpec((EMBED, EMBED), full),
            pl.BlockSpec((1, EMBED), full),
            pl.BlockSpec((EMBED, EMBED), full),
            pl.BlockSpec((1, EMBED), full),
        ],
        out_specs=[
            pl.BlockSpec((NB, EMBED), blk),
            pl.BlockSpec((NB, EMBED), blk),
            pl.BlockSpec((NB, EMBED), blk),
        ],
        out_shape=[jax.ShapeDtypeStruct((N, EMBED), jnp.float32)] * 3,
    )(nf, gf, wsk, bsk, wsr, bsr, wd, bd)


def _ke_body(ef_ref, we_ref, be_ref, out_ref):
    out_ref[...] = (
        jnp.dot(ef_ref[...], we_ref[...], preferred_element_type=jnp.float32)
        + be_ref[...])


def _proj_edges(ef, we, be):
    return pl.pallas_call(
        _ke_body,
        grid=(E // EB,),
        in_specs=[
            pl.BlockSpec((EB, 16), lambda i: (i, 0)),
            pl.BlockSpec((16, EMBED), lambda i: (0, 0)),
            pl.BlockSpec((1, EMBED), lambda i: (0, 0)),
        ],
        out_specs=pl.BlockSpec((EB, EMBED), lambda i: (i, 0)),
        out_shape=jax.ShapeDtypeStruct((E, EMBED), jnp.float32),
    )(ef, we, be)


def _kb_body(agg_ref, skip_ref, ln1s_ref, ln1b_ref, wk_ref, bk_ref, wv_ref,
             bv_ref, gf_ref, wq_ref, bq_ref, wo_ref, bo_ref,
             nf2_ref, gfn_ref, q_s, m_s, l_s, o_s):
    i = pl.program_id(0)

    @pl.when(i == 0)
    def _init():
        q_s[...] = (jnp.dot(gf_ref[...], wq_ref[...],
                            preferred_element_type=jnp.float32) + bq_ref[...])
        m_s[...] = jnp.full((1, HEADS), -jnp.inf, jnp.float32)
        l_s[...] = jnp.zeros((1, HEADS), jnp.float32)
        o_s[...] = jnp.zeros((1, EMBED), jnp.float32)

    x = agg_ref[...] + skip_ref[...]
    mu = jnp.mean(x, axis=-1, keepdims=True)
    var = jnp.mean((x - mu) ** 2, axis=-1, keepdims=True)
    nf2 = (x - mu) / jnp.sqrt(var + 1e-6) * ln1s_ref[...] + ln1b_ref[...]
    nf2 = jnp.maximum(nf2, 0.0)
    nf2_ref[...] = nf2

    k = jnp.dot(nf2, wk_ref[...], preferred_element_type=jnp.float32) + bk_ref[...]
    v = jnp.dot(nf2, wv_ref[...], preferred_element_type=jnp.float32) + bv_ref[...]

    d_ids = jax.lax.broadcasted_iota(jnp.int32, (EMBED, HEADS), 0) // DH
    h_ids = jax.lax.broadcasted_iota(jnp.int32, (EMBED, HEADS), 1)
    sel = (d_ids == h_ids).astype(jnp.float32)
    qm = q_s[...].reshape(EMBED, 1) * sel
    logits = jnp.dot(k, qm, preferred_element_type=jnp.float32) * _INV_SQRT_DH

    m_prev = m_s[...]
    m_blk = jnp.max(logits, axis=0, keepdims=True)
    m_new = jnp.maximum(m_prev, m_blk)
    corr = jnp.exp(m_prev - m_new)
    w = jnp.exp(logits - m_new)
    h2d = (jax.lax.broadcasted_iota(jnp.int32, (HEADS, EMBED), 0)
           == jax.lax.broadcasted_iota(jnp.int32, (HEADS, EMBED), 1) // DH
           ).astype(jnp.float32)
    w_exp = jnp.dot(w, h2d, preferred_element_type=jnp.float32)
    corr_exp = jnp.dot(corr, h2d, preferred_element_type=jnp.float32)
    m_s[...] = m_new
    l_s[...] = l_s[...] * corr + jnp.sum(w, axis=0, keepdims=True)
    o_s[...] = o_s[...] * corr_exp + jnp.sum(w_exp * v, axis=0, keepdims=True)

    @pl.when(i == pl.num_programs(0) - 1)
    def _fin():
        l_exp = jnp.dot(l_s[...], h2d, preferred_element_type=jnp.float32)
        attn = o_s[...] / l_exp
        gfn_ref[...] = gf_ref[...] + (
            jnp.dot(attn, wo_ref[...], preferred_element_type=jnp.float32)
            + bo_ref[...])


def _post_nodes(agg, skip, ln1s, ln1b, wk, bk, wv, bv, gf, wq, bq, wo, bo):
    full = lambda i: (0, 0)
    blk = lambda i: (i, 0)
    return pl.pallas_call(
        _kb_body,
        grid=(N // NB,),
        in_specs=[
            pl.BlockSpec((NB, EMBED), blk),
            pl.BlockSpec((NB, EMBED), blk),
            pl.BlockSpec((1, EMBED), full),
            pl.BlockSpec((1, EMBED), full),
            pl.BlockSpec((EMBED, EMBED), full),
            pl.BlockSpec((1, EMBED), full),
            pl.BlockSpec((EMBED, EMBED), full),
            pl.BlockSpec((1, EMBED), full),
            pl.BlockSpec((1, EMBED), full),
            pl.BlockSpec((EMBED, EMBED), full),
            pl.BlockSpec((1, EMBED), full),
            pl.BlockSpec((EMBED, EMBED), full),
            pl.BlockSpec((1, EMBED), full),
        ],
        out_specs=[
            pl.BlockSpec((NB, EMBED), blk),
            pl.BlockSpec((1, EMBED), full),
        ],
        out_shape=[
            jax.ShapeDtypeStruct((N, EMBED), jnp.float32),
            jax.ShapeDtypeStruct((1, EMBED), jnp.float32),
        ],
        scratch_shapes=[
            pltpu.VMEM((1, EMBED), jnp.float32),
            pltpu.VMEM((1, HEADS), jnp.float32),
            pltpu.VMEM((1, HEADS), jnp.float32),
            pltpu.VMEM((1, EMBED), jnp.float32),
        ],
    )(agg, skip, ln1s, ln1b, wk, bk, wv, bv, gf, wq, bq, wo, bo)


def _kf_body(gf_ref, s_ref, b_ref, out_ref):
    x = gf_ref[...]
    mu = jnp.mean(x, axis=-1, keepdims=True)
    var = jnp.mean((x - mu) ** 2, axis=-1, keepdims=True)
    out_ref[...] = (x - mu) / jnp.sqrt(var + 1e-6) * s_ref[...] + b_ref[...]


def _final_ln(gf, s, b):
    return pl.pallas_call(
        _kf_body,
        out_shape=jax.ShapeDtypeStruct((1, EMBED), jnp.float32),
    )(gf, s, b)


def _edge_phase(xs, xd, ep, senders, receivers, attn_a):
    feat = (xs[senders] + xd[receivers] + ep).reshape(E, HEADS, DH)
    score = jnp.sum(jnp.where(feat > 0, feat, 0.2 * feat) * attn_a, axis=-1)
    smax = jax.ops.segment_max(score, receivers, num_segments=N)
    smax = jnp.where(jnp.isfinite(smax), smax, 0.0)
    exps = jnp.exp(score - smax[receivers])
    denom = jax.ops.segment_sum(exps, receivers, num_segments=N)
    msg = xs[senders].reshape(E, HEADS, DH) * exps[..., None]
    wsum = jax.ops.segment_sum(msg, receivers, num_segments=N)
    return (wsum / (denom + 1e-9)[..., None]).reshape(N, EMBED)


def kernel(node_features, edge_features, global_features, edge_list, W_skip,
           b_skip, W_src, b_src, W_dst, b_dst, W_edge, b_edge, attn_a,
           ln1_scale, ln1_bias, Wq, bq, Wk, bk, Wv, bv, Wo, bo, global_param,
           lnf_scale, lnf_bias):
    senders = edge_list[:, 0]
    receivers = edge_list[:, 1]
    nf = jnp.concatenate(
        [node_features, jnp.broadcast_to(global_features, (N, 128))], axis=-1)
    gf = global_param

    r2 = lambda a: a.reshape(1, EMBED)
    for i in range(LAYERS):
        skip, xs, xd = _proj_nodes(nf, gf, W_skip[i], r2(b_skip[i]),
                                   W_src[i], r2(b_src[i]),
                                   W_dst[i], r2(b_dst[i]))
        ep = _proj_edges(edge_features, W_edge[i], r2(b_edge[i]))
        agg = _edge_phase(xs, xd, ep, senders, receivers, attn_a[i])
        nf, gf = _post_nodes(agg, skip, r2(ln1_scale[i]), r2(ln1_bias[i]),
                             Wk[i], r2(bk[i]), Wv[i], r2(bv[i]), gf,
                             Wq[i], r2(bq[i]), Wo[i], r2(bo[i]))
    gf = _final_ln(gf, lnf_scale.reshape(1, EMBED), lnf_bias.reshape(1, EMBED))
    return gf.reshape(-1)
